# Initial kernel scaffold; baseline (speedup 1.0000x reference)
import jax
import jax.numpy as jnp
from jax.experimental import pallas as pl


def kernel(x, edge_index, W_conv, b_conv, W_lin, b_lin, W_gate, b_gate, bn_gamma, bn_beta):
    raise NotImplementedError("write your pallas kernel here")


# speedup vs baseline: 15.7767x; 15.7767x over previous
import functools

import jax
import jax.numpy as jnp
from jax import lax
from jax.experimental import pallas as pl
from jax.experimental.pallas import tpu as pltpu
from jax.experimental.pallas import tpu_sc as plsc

N = 10000
D = 128
H = 128
E = 320000

NC = 2
NS = 16
LW = 128
R = 80
E_PAD = NC * NS * R * LW
N_PAD = 10112
ROWS_PER_TILE = N_PAD // NS

@functools.cache
def _sc_mesh():
  return plsc.VectorSubcoreMesh(
      core_axis_name="c", subcore_axis_name="s", num_cores=NC, num_subcores=NS)


_BUF = 640


def _deg_body(dst_hbm, out_hbm, deg_s, idx_v, ones_v, buf_v):
  c = lax.axis_index("c")
  s = lax.axis_index("s")
  wid = c * NS + s
  w0 = s * ROWS_PER_TILE
  for k in range(LW // 16):
    ones_v[pl.ds(k * 16, 16)] = jnp.ones((16,), jnp.float32)
  for k in range(_BUF // 16):
    buf_v[pl.ds(k * 16, 16)] = jnp.zeros((16,), jnp.float32)
  pltpu.sync_copy(buf_v.at[pl.ds(0, ROWS_PER_TILE)],
                  deg_s.at[pl.ds(w0, ROWS_PER_TILE)])
  pltpu.sync_copy(dst_hbm.at[pl.ds(wid * R, R)], idx_v)
  plsc.subcore_barrier()

  @pl.loop(0, R)
  def _(j):
    pltpu.sync_copy(ones_v, deg_s.at[idx_v.at[j]], add=True)

  plsc.subcore_barrier()
  pltpu.sync_copy(deg_s.at[pl.ds(w0, ROWS_PER_TILE)],
                  buf_v.at[pl.ds(0, ROWS_PER_TILE)])
  pltpu.sync_copy(buf_v.at[pl.ds(0, ROWS_PER_TILE)],
                  out_hbm.at[pl.ds(c * N_PAD + w0, ROWS_PER_TILE)])


@functools.cache
def _deg_call():
  return pl.kernel(
      _deg_body,
      out_type=jax.ShapeDtypeStruct((NC * N_PAD,), jnp.float32),
      mesh=_sc_mesh(),
      scratch_types=[
          pltpu.VMEM_SHARED((N_PAD,), jnp.float32),
          pltpu.VMEM((R, LW), jnp.int32),
          pltpu.VMEM((LW,), jnp.float32),
          pltpu.VMEM((_BUF,), jnp.float32),
      ],
  )


HALF = R // 2


def _gather_scatter_body(y_hbm, src_hbm, dst_hbm, zeros_hbm, out_hbm,
                         acc_s, src_v, dst_v, rows0, rows1, sem0, sem1):
  c = lax.axis_index("c")
  s = lax.axis_index("s")
  wid = c * NS + s
  base = wid * R
  row0 = s * ROWS_PER_TILE
  pltpu.sync_copy(zeros_hbm.at[pl.ds(row0, ROWS_PER_TILE)],
                  acc_s.at[pl.ds(row0, ROWS_PER_TILE)])
  plsc.subcore_barrier()

  @pl.loop(0, 2)
  def _(h):
    hb = base + h * HALF
    pltpu.sync_copy(src_hbm.at[pl.ds(hb, HALF)], src_v)
    pltpu.sync_copy(dst_hbm.at[pl.ds(hb, HALF)], dst_v)

    pltpu.async_copy(y_hbm.at[src_v.at[0]], rows0, sem0)

    @pl.loop(0, HALF, step=2)
    def _(j):
      pltpu.make_async_copy(y_hbm.at[src_v.at[j]], rows0, sem0).wait()
      pltpu.async_copy(y_hbm.at[src_v.at[j + 1]], rows1, sem1)
      pltpu.sync_copy(rows0, acc_s.at[dst_v.at[j]], add=True)
      pltpu.make_async_copy(y_hbm.at[src_v.at[j + 1]], rows1, sem1).wait()

      @pl.when(j + 2 < HALF)
      def _():
        pltpu.async_copy(y_hbm.at[src_v.at[j + 2]], rows0, sem0)

      pltpu.sync_copy(rows1, acc_s.at[dst_v.at[j + 1]], add=True)

  plsc.subcore_barrier()
  pltpu.sync_copy(acc_s.at[pl.ds(row0, ROWS_PER_TILE)],
                  out_hbm.at[c, pl.ds(row0, ROWS_PER_TILE)])


@functools.cache
def _gather_scatter_call():
  return pl.kernel(
      _gather_scatter_body,
      out_type=jax.ShapeDtypeStruct((NC, N_PAD, H), jnp.float32),
      mesh=_sc_mesh(),
      scratch_types=[
          pltpu.VMEM_SHARED((N_PAD, H), jnp.float32),
          pltpu.VMEM((HALF, LW), jnp.int32),
          pltpu.VMEM((HALF, LW), jnp.int32),
          pltpu.VMEM((LW, H), jnp.float32),
          pltpu.VMEM((LW, H), jnp.float32),
          pltpu.SemaphoreType.DMA,
          pltpu.SemaphoreType.DMA,
      ],
  )


BLK = 1000


def _dis_from_deg(degA, degB):
  return lax.rsqrt(degA + degB + 1.0)


def _y_body(x_ref, w_ref, degA_ref, degB_ref, y_ref):
  dis = _dis_from_deg(degA_ref[...], degB_ref[...])
  xw = lax.dot_general(x_ref[...], w_ref[...], (((1,), (1,)), ((), ())),
                       preferred_element_type=jnp.float32)
  y_ref[...] = xw * dis


def _fuse_body(accA_ref, accB_ref, y_ref, degA_ref, degB_ref, x_ref,
               wg_ref, bg_ref, wl_ref, bl_ref, bc_ref,
               out_ref, sum_ref, sumsq_ref):
  i = pl.program_id(0)
  dis = _dis_from_deg(degA_ref[...], degB_ref[...])
  y = y_ref[...]
  z = jnp.tanh((accA_ref[...] + accB_ref[...] + y) * dis + bc_ref[...])
  g = jax.nn.sigmoid(
      lax.dot_general(z, wg_ref[...], (((1,), (1,)), ((), ())),
                      preferred_element_type=jnp.float32) + bg_ref[...])
  xl = lax.dot_general(x_ref[...], wl_ref[...], (((1,), (1,)), ((), ())),
                       preferred_element_type=jnp.float32) + bl_ref[...]
  o = jnp.maximum((1.0 - g) * xl + g * z, 0.0)
  out_ref[...] = o
  ps = jnp.sum(o.reshape(BLK // 8, 8, H), axis=0)
  pq = jnp.sum((o * o).reshape(BLK // 8, 8, H), axis=0)

  @pl.when(i == 0)
  def _():
    sum_ref[...] = jnp.zeros_like(sum_ref)
    sumsq_ref[...] = jnp.zeros_like(sumsq_ref)

  sum_ref[...] += ps
  sumsq_ref[...] += pq


def _bn_body(out_pre_ref, sum_ref, sumsq_ref, gamma_ref, beta_ref, out_ref):
  mean = jnp.sum(sum_ref[...], axis=0) * (1.0 / N)
  ex2 = jnp.sum(sumsq_ref[...], axis=0) * (1.0 / N)
  var = ex2 - mean * mean
  scale = lax.rsqrt(var + 1e-5) * gamma_ref[0]
  shift = beta_ref[0] - mean * scale
  out_ref[...] = out_pre_ref[...] * scale[None, :] + shift[None, :]


def _row_spec(width):
  return pl.BlockSpec((BLK, width), lambda i: (i, 0))


def _full_spec(shape):
  return pl.BlockSpec(shape, lambda i: tuple(0 for _ in shape))


@jax.jit
def kernel(x, edge_index, W_conv, b_conv, W_lin, b_lin, W_gate, b_gate,
           bn_gamma, bn_beta):
  src = edge_index[0]
  dst = edge_index[1]
  pad = E_PAD - E
  pad_dst = N + (jnp.arange(pad, dtype=jnp.int32) % (N_PAD - N))
  src_p = jnp.concatenate([src, jnp.zeros((pad,), jnp.int32)]).reshape(
      NC * NS * R, LW)
  dst_p = jnp.concatenate([dst, pad_dst]).reshape(NC * NS * R, LW)
  zerosN = jnp.zeros((N_PAD, H), jnp.float32)

  deg = _deg_call()(dst_p).reshape(NC, N_PAD)
  degA = deg[0, :N].reshape(N, 1)
  degB = deg[1, :N].reshape(N, 1)

  y = pl.pallas_call(
      _y_body,
      grid=(N // BLK,),
      in_specs=[_row_spec(D), _full_spec((H, D)), _row_spec(1),
                _row_spec(1)],
      out_specs=_row_spec(H),
      out_shape=jax.ShapeDtypeStruct((N, H), jnp.float32),
  )(x, W_conv, degA, degB)

  accs = _gather_scatter_call()(y, src_p, dst_p, zerosN)
  accA = accs[0, :N]
  accB = accs[1, :N]

  bc2 = b_conv.reshape(1, H)
  bg2 = b_gate.reshape(1, H)
  bl2 = b_lin.reshape(1, H)

  out_pre, s8, q8 = pl.pallas_call(
      _fuse_body,
      grid=(N // BLK,),
      in_specs=[_row_spec(H), _row_spec(H), _row_spec(H), _row_spec(1),
                _row_spec(1), _row_spec(D), _full_spec((H, H)),
                _full_spec((1, H)), _full_spec((H, D)), _full_spec((1, H)),
                _full_spec((1, H))],
      out_specs=[_row_spec(H), _full_spec((8, H)), _full_spec((8, H))],
      out_shape=[jax.ShapeDtypeStruct((N, H), jnp.float32),
                 jax.ShapeDtypeStruct((8, H), jnp.float32),
                 jax.ShapeDtypeStruct((8, H), jnp.float32)],
  )(accA, accB, y, degA, degB, x, W_gate, bg2, W_lin, bl2, bc2)

  out = pl.pallas_call(
      _bn_body,
      grid=(N // BLK,),
      in_specs=[_row_spec(H), _full_spec((8, H)), _full_spec((8, H)),
                _full_spec((1, H)), _full_spec((1, H))],
      out_specs=_row_spec(H),
      out_shape=jax.ShapeDtypeStruct((N, H), jnp.float32),
  )(out_pre, s8, q8, bn_gamma.reshape(1, H), bn_beta.reshape(1, H))
  return out


# speedup vs baseline: 16.4665x; 1.0437x over previous
import functools

import jax
import jax.numpy as jnp
from jax import lax
from jax.experimental import pallas as pl
from jax.experimental.pallas import tpu as pltpu
from jax.experimental.pallas import tpu_sc as plsc

N = 10000
D = 128
H = 128
E = 320000

NC = 2
NS = 16
LW = 128
R = 80
E_PAD = NC * NS * R * LW
N_PAD = 10112
ROWS_PER_TILE = N_PAD // NS

@functools.cache
def _sc_mesh():
  return plsc.VectorSubcoreMesh(
      core_axis_name="c", subcore_axis_name="s", num_cores=NC, num_subcores=NS)


_BUF = 640


def _deg_body(dst_hbm, out_hbm, deg_s, idx_v, ones_v, buf_v):
  c = lax.axis_index("c")
  s = lax.axis_index("s")
  wid = c * NS + s
  w0 = s * ROWS_PER_TILE
  for k in range(LW // 16):
    ones_v[pl.ds(k * 16, 16)] = jnp.ones((16,), jnp.float32)
  for k in range(_BUF // 16):
    buf_v[pl.ds(k * 16, 16)] = jnp.zeros((16,), jnp.float32)
  pltpu.sync_copy(buf_v.at[pl.ds(0, ROWS_PER_TILE)],
                  deg_s.at[pl.ds(w0, ROWS_PER_TILE)])
  pltpu.sync_copy(dst_hbm.at[pl.ds(wid * R, R)], idx_v)
  plsc.subcore_barrier()

  @pl.loop(0, R)
  def _(j):
    pltpu.sync_copy(ones_v, deg_s.at[idx_v.at[j]], add=True)

  plsc.subcore_barrier()
  pltpu.sync_copy(deg_s.at[pl.ds(w0, ROWS_PER_TILE)],
                  buf_v.at[pl.ds(0, ROWS_PER_TILE)])
  pltpu.sync_copy(buf_v.at[pl.ds(0, ROWS_PER_TILE)],
                  out_hbm.at[pl.ds(c * N_PAD + w0, ROWS_PER_TILE)])


@functools.cache
def _deg_call():
  return pl.kernel(
      _deg_body,
      out_type=jax.ShapeDtypeStruct((NC * N_PAD,), jnp.float32),
      mesh=_sc_mesh(),
      scratch_types=[
          pltpu.VMEM_SHARED((N_PAD,), jnp.float32),
          pltpu.VMEM((R, LW), jnp.int32),
          pltpu.VMEM((LW,), jnp.float32),
          pltpu.VMEM((_BUF,), jnp.float32),
      ],
  )


HALF = R // 2


_GS = LW // 2


def _issue_gather(y_hbm, src_v, j, buf, s0, s1):
  pltpu.async_copy(y_hbm.at[src_v.at[j, pl.ds(0, _GS)]],
                   buf.at[pl.ds(0, _GS)], s0)
  pltpu.async_copy(y_hbm.at[src_v.at[j, pl.ds(_GS, _GS)]],
                   buf.at[pl.ds(_GS, _GS)], s1)


def _wait_gather(y_hbm, src_v, j, buf, s0, s1):
  pltpu.make_async_copy(y_hbm.at[src_v.at[j, pl.ds(0, _GS)]],
                        buf.at[pl.ds(0, _GS)], s0).wait()
  pltpu.make_async_copy(y_hbm.at[src_v.at[j, pl.ds(_GS, _GS)]],
                        buf.at[pl.ds(_GS, _GS)], s1).wait()


def _gather_scatter_body(y_hbm, src_hbm, dst_hbm, zeros_hbm, out_hbm,
                         acc_s, src_v, dst_v, rows0, rows1,
                         sa0, sa1, sb0, sb1):
  c = lax.axis_index("c")
  s = lax.axis_index("s")
  wid = c * NS + s
  base = wid * R
  row0 = s * ROWS_PER_TILE
  pltpu.sync_copy(zeros_hbm.at[pl.ds(row0, ROWS_PER_TILE)],
                  acc_s.at[pl.ds(row0, ROWS_PER_TILE)])
  plsc.subcore_barrier()

  @pl.loop(0, 2)
  def _(h):
    hb = base + h * HALF
    pltpu.sync_copy(src_hbm.at[pl.ds(hb, HALF)], src_v)
    pltpu.sync_copy(dst_hbm.at[pl.ds(hb, HALF)], dst_v)

    _issue_gather(y_hbm, src_v, 0, rows0, sa0, sa1)

    @pl.loop(0, HALF, step=2)
    def _(j):
      _issue_gather(y_hbm, src_v, j + 1, rows1, sb0, sb1)
      _wait_gather(y_hbm, src_v, j, rows0, sa0, sa1)
      pltpu.sync_copy(rows0, acc_s.at[dst_v.at[j]], add=True)

      @pl.when(j + 2 < HALF)
      def _():
        _issue_gather(y_hbm, src_v, j + 2, rows0, sa0, sa1)

      _wait_gather(y_hbm, src_v, j + 1, rows1, sb0, sb1)
      pltpu.sync_copy(rows1, acc_s.at[dst_v.at[j + 1]], add=True)

  plsc.subcore_barrier()
  pltpu.sync_copy(acc_s.at[pl.ds(row0, ROWS_PER_TILE)],
                  out_hbm.at[c, pl.ds(row0, ROWS_PER_TILE)])


@functools.cache
def _gather_scatter_call():
  return pl.kernel(
      _gather_scatter_body,
      out_type=jax.ShapeDtypeStruct((NC, N_PAD, H), jnp.float32),
      mesh=_sc_mesh(),
      scratch_types=[
          pltpu.VMEM_SHARED((N_PAD, H), jnp.float32),
          pltpu.VMEM((HALF, LW), jnp.int32),
          pltpu.VMEM((HALF, LW), jnp.int32),
          pltpu.VMEM((LW, H), jnp.float32),
          pltpu.VMEM((LW, H), jnp.float32),
          pltpu.SemaphoreType.DMA,
          pltpu.SemaphoreType.DMA,
          pltpu.SemaphoreType.DMA,
          pltpu.SemaphoreType.DMA,
      ],
  )


BLK = 1000


def _dis_from_deg(degA, degB):
  return lax.rsqrt(degA + degB + 1.0)


def _y_body(x_ref, w_ref, degA_ref, degB_ref, y_ref):
  dis = _dis_from_deg(degA_ref[...], degB_ref[...])
  xw = lax.dot_general(x_ref[...], w_ref[...], (((1,), (1,)), ((), ())),
                       preferred_element_type=jnp.float32)
  y_ref[...] = xw * dis


def _fuse_body(accA_ref, accB_ref, y_ref, degA_ref, degB_ref, x_ref,
               wg_ref, bg_ref, wl_ref, bl_ref, bc_ref,
               out_ref, sum_ref, sumsq_ref):
  i = pl.program_id(0)
  dis = _dis_from_deg(degA_ref[...], degB_ref[...])
  y = y_ref[...]
  z = jnp.tanh((accA_ref[...] + accB_ref[...] + y) * dis + bc_ref[...])
  g = jax.nn.sigmoid(
      lax.dot_general(z, wg_ref[...], (((1,), (1,)), ((), ())),
                      preferred_element_type=jnp.float32) + bg_ref[...])
  xl = lax.dot_general(x_ref[...], wl_ref[...], (((1,), (1,)), ((), ())),
                       preferred_element_type=jnp.float32) + bl_ref[...]
  o = jnp.maximum((1.0 - g) * xl + g * z, 0.0)
  out_ref[...] = o
  ps = jnp.sum(o.reshape(BLK // 8, 8, H), axis=0)
  pq = jnp.sum((o * o).reshape(BLK // 8, 8, H), axis=0)

  @pl.when(i == 0)
  def _():
    sum_ref[...] = jnp.zeros_like(sum_ref)
    sumsq_ref[...] = jnp.zeros_like(sumsq_ref)

  sum_ref[...] += ps
  sumsq_ref[...] += pq


def _bn_body(out_pre_ref, sum_ref, sumsq_ref, gamma_ref, beta_ref, out_ref):
  mean = jnp.sum(sum_ref[...], axis=0) * (1.0 / N)
  ex2 = jnp.sum(sumsq_ref[...], axis=0) * (1.0 / N)
  var = ex2 - mean * mean
  scale = lax.rsqrt(var + 1e-5) * gamma_ref[0]
  shift = beta_ref[0] - mean * scale
  out_ref[...] = out_pre_ref[...] * scale[None, :] + shift[None, :]


def _row_spec(width):
  return pl.BlockSpec((BLK, width), lambda i: (i, 0))


def _full_spec(shape):
  return pl.BlockSpec(shape, lambda i: tuple(0 for _ in shape))


@jax.jit
def kernel(x, edge_index, W_conv, b_conv, W_lin, b_lin, W_gate, b_gate,
           bn_gamma, bn_beta):
  src = edge_index[0]
  dst = edge_index[1]
  pad = E_PAD - E
  pad_dst = N + (jnp.arange(pad, dtype=jnp.int32) % (N_PAD - N))
  src_p = jnp.concatenate([src, jnp.zeros((pad,), jnp.int32)]).reshape(
      NC * NS * R, LW)
  dst_p = jnp.concatenate([dst, pad_dst]).reshape(NC * NS * R, LW)
  zerosN = jnp.zeros((N_PAD, H), jnp.float32)

  deg = _deg_call()(dst_p).reshape(NC, N_PAD)
  degA = deg[0, :N].reshape(N, 1)
  degB = deg[1, :N].reshape(N, 1)

  y = pl.pallas_call(
      _y_body,
      grid=(N // BLK,),
      in_specs=[_row_spec(D), _full_spec((H, D)), _row_spec(1),
                _row_spec(1)],
      out_specs=_row_spec(H),
      out_shape=jax.ShapeDtypeStruct((N, H), jnp.float32),
  )(x, W_conv, degA, degB)

  accs = _gather_scatter_call()(y, src_p, dst_p, zerosN)
  accA = accs[0, :N]
  accB = accs[1, :N]

  bc2 = b_conv.reshape(1, H)
  bg2 = b_gate.reshape(1, H)
  bl2 = b_lin.reshape(1, H)

  out_pre, s8, q8 = pl.pallas_call(
      _fuse_body,
      grid=(N // BLK,),
      in_specs=[_row_spec(H), _row_spec(H), _row_spec(H), _row_spec(1),
                _row_spec(1), _row_spec(D), _full_spec((H, H)),
                _full_spec((1, H)), _full_spec((H, D)), _full_spec((1, H)),
                _full_spec((1, H))],
      out_specs=[_row_spec(H), _full_spec((8, H)), _full_spec((8, H))],
      out_shape=[jax.ShapeDtypeStruct((N, H), jnp.float32),
                 jax.ShapeDtypeStruct((8, H), jnp.float32),
                 jax.ShapeDtypeStruct((8, H), jnp.float32)],
  )(accA, accB, y, degA, degB, x, W_gate, bg2, W_lin, bl2, bc2)

  out = pl.pallas_call(
      _bn_body,
      grid=(N // BLK,),
      in_specs=[_row_spec(H), _full_spec((8, H)), _full_spec((8, H)),
                _full_spec((1, H)), _full_spec((1, H))],
      out_specs=_row_spec(H),
      out_shape=jax.ShapeDtypeStruct((N, H), jnp.float32),
  )(out_pre, s8, q8, bn_gamma.reshape(1, H), bn_beta.reshape(1, H))
  return out


# speedup vs baseline: 16.4733x; 1.0004x over previous
import functools

import jax
import jax.numpy as jnp
from jax import lax
from jax.experimental import pallas as pl
from jax.experimental.pallas import tpu as pltpu
from jax.experimental.pallas import tpu_sc as plsc

N = 10000
D = 128
H = 128
E = 320000

NC = 2
NS = 16
LW = 128
R = 80
E_PAD = NC * NS * R * LW
N_PAD = 10112
ROWS_PER_TILE = N_PAD // NS

@functools.cache
def _sc_mesh():
  return plsc.VectorSubcoreMesh(
      core_axis_name="c", subcore_axis_name="s", num_cores=NC, num_subcores=NS)


_BUF = 640


def _deg_body(dst_hbm, out_hbm, deg_s, idx_v, ones_v, buf_v):
  c = lax.axis_index("c")
  s = lax.axis_index("s")
  wid = c * NS + s
  w0 = s * ROWS_PER_TILE
  for k in range(LW // 16):
    ones_v[pl.ds(k * 16, 16)] = jnp.ones((16,), jnp.float32)
  for k in range(_BUF // 16):
    buf_v[pl.ds(k * 16, 16)] = jnp.zeros((16,), jnp.float32)
  pltpu.sync_copy(buf_v.at[pl.ds(0, ROWS_PER_TILE)],
                  deg_s.at[pl.ds(w0, ROWS_PER_TILE)])
  pltpu.sync_copy(dst_hbm.at[pl.ds(wid * R, R)], idx_v)
  plsc.subcore_barrier()

  @pl.loop(0, R)
  def _(j):
    pltpu.sync_copy(ones_v, deg_s.at[idx_v.at[j]], add=True)

  plsc.subcore_barrier()
  pltpu.sync_copy(deg_s.at[pl.ds(w0, ROWS_PER_TILE)],
                  buf_v.at[pl.ds(0, ROWS_PER_TILE)])
  pltpu.sync_copy(buf_v.at[pl.ds(0, ROWS_PER_TILE)],
                  out_hbm.at[pl.ds(c * N_PAD + w0, ROWS_PER_TILE)])


@functools.cache
def _deg_call():
  return pl.kernel(
      _deg_body,
      out_type=jax.ShapeDtypeStruct((NC * N_PAD,), jnp.float32),
      mesh=_sc_mesh(),
      scratch_types=[
          pltpu.VMEM_SHARED((N_PAD,), jnp.float32),
          pltpu.VMEM((R, LW), jnp.int32),
          pltpu.VMEM((LW,), jnp.float32),
          pltpu.VMEM((_BUF,), jnp.float32),
      ],
  )


HALF = R // 2


_GS = LW // 4


def _issue_gather(y_hbm, src_v, j, buf, s0, s1):
  for k, sem in ((0, s0), (1, s0), (2, s1), (3, s1)):
    pltpu.async_copy(y_hbm.at[src_v.at[j, pl.ds(k * _GS, _GS)]],
                     buf.at[pl.ds(k * _GS, _GS)], sem)


def _wait_gather(y_hbm, src_v, j, buf, s0, s1):
  for k, sem in ((0, s0), (1, s0), (2, s1), (3, s1)):
    pltpu.make_async_copy(y_hbm.at[src_v.at[j, pl.ds(k * _GS, _GS)]],
                          buf.at[pl.ds(k * _GS, _GS)], sem).wait()


def _gather_scatter_body(y_hbm, src_hbm, dst_hbm, zeros_hbm, out_hbm,
                         acc_s, src_v, dst_v, rows0, rows1,
                         sa0, sa1, sb0, sb1):
  c = lax.axis_index("c")
  s = lax.axis_index("s")
  wid = c * NS + s
  base = wid * R
  row0 = s * ROWS_PER_TILE
  pltpu.sync_copy(zeros_hbm.at[pl.ds(row0, ROWS_PER_TILE)],
                  acc_s.at[pl.ds(row0, ROWS_PER_TILE)])
  plsc.subcore_barrier()

  @pl.loop(0, 2)
  def _(h):
    hb = base + h * HALF
    pltpu.sync_copy(src_hbm.at[pl.ds(hb, HALF)], src_v)
    pltpu.sync_copy(dst_hbm.at[pl.ds(hb, HALF)], dst_v)

    _issue_gather(y_hbm, src_v, 0, rows0, sa0, sa1)

    @pl.loop(0, HALF, step=2)
    def _(j):
      _issue_gather(y_hbm, src_v, j + 1, rows1, sb0, sb1)
      _wait_gather(y_hbm, src_v, j, rows0, sa0, sa1)
      pltpu.sync_copy(rows0, acc_s.at[dst_v.at[j]], add=True)

      @pl.when(j + 2 < HALF)
      def _():
        _issue_gather(y_hbm, src_v, j + 2, rows0, sa0, sa1)

      _wait_gather(y_hbm, src_v, j + 1, rows1, sb0, sb1)
      pltpu.sync_copy(rows1, acc_s.at[dst_v.at[j + 1]], add=True)

  plsc.subcore_barrier()
  pltpu.sync_copy(acc_s.at[pl.ds(row0, ROWS_PER_TILE)],
                  out_hbm.at[c, pl.ds(row0, ROWS_PER_TILE)])


@functools.cache
def _gather_scatter_call():
  return pl.kernel(
      _gather_scatter_body,
      out_type=jax.ShapeDtypeStruct((NC, N_PAD, H), jnp.float32),
      mesh=_sc_mesh(),
      scratch_types=[
          pltpu.VMEM_SHARED((N_PAD, H), jnp.float32),
          pltpu.VMEM((HALF, LW), jnp.int32),
          pltpu.VMEM((HALF, LW), jnp.int32),
          pltpu.VMEM((LW, H), jnp.float32),
          pltpu.VMEM((LW, H), jnp.float32),
          pltpu.SemaphoreType.DMA,
          pltpu.SemaphoreType.DMA,
          pltpu.SemaphoreType.DMA,
          pltpu.SemaphoreType.DMA,
      ],
  )


BLK = 1000


def _dis_from_deg(degA, degB):
  return lax.rsqrt(degA + degB + 1.0)


def _y_body(x_ref, w_ref, degA_ref, degB_ref, y_ref):
  dis = _dis_from_deg(degA_ref[...], degB_ref[...])
  xw = lax.dot_general(x_ref[...], w_ref[...], (((1,), (1,)), ((), ())),
                       preferred_element_type=jnp.float32)
  y_ref[...] = xw * dis


def _fuse_body(accA_ref, accB_ref, y_ref, degA_ref, degB_ref, x_ref,
               wg_ref, bg_ref, wl_ref, bl_ref, bc_ref,
               out_ref, sum_ref, sumsq_ref):
  i = pl.program_id(0)
  dis = _dis_from_deg(degA_ref[...], degB_ref[...])
  y = y_ref[...]
  z = jnp.tanh((accA_ref[...] + accB_ref[...] + y) * dis + bc_ref[...])
  g = jax.nn.sigmoid(
      lax.dot_general(z, wg_ref[...], (((1,), (1,)), ((), ())),
                      preferred_element_type=jnp.float32) + bg_ref[...])
  xl = lax.dot_general(x_ref[...], wl_ref[...], (((1,), (1,)), ((), ())),
                       preferred_element_type=jnp.float32) + bl_ref[...]
  o = jnp.maximum((1.0 - g) * xl + g * z, 0.0)
  out_ref[...] = o
  ps = jnp.sum(o.reshape(BLK // 8, 8, H), axis=0)
  pq = jnp.sum((o * o).reshape(BLK // 8, 8, H), axis=0)

  @pl.when(i == 0)
  def _():
    sum_ref[...] = jnp.zeros_like(sum_ref)
    sumsq_ref[...] = jnp.zeros_like(sumsq_ref)

  sum_ref[...] += ps
  sumsq_ref[...] += pq


def _bn_body(out_pre_ref, sum_ref, sumsq_ref, gamma_ref, beta_ref, out_ref):
  mean = jnp.sum(sum_ref[...], axis=0) * (1.0 / N)
  ex2 = jnp.sum(sumsq_ref[...], axis=0) * (1.0 / N)
  var = ex2 - mean * mean
  scale = lax.rsqrt(var + 1e-5) * gamma_ref[0]
  shift = beta_ref[0] - mean * scale
  out_ref[...] = out_pre_ref[...] * scale[None, :] + shift[None, :]


def _row_spec(width):
  return pl.BlockSpec((BLK, width), lambda i: (i, 0))


def _full_spec(shape):
  return pl.BlockSpec(shape, lambda i: tuple(0 for _ in shape))


@jax.jit
def kernel(x, edge_index, W_conv, b_conv, W_lin, b_lin, W_gate, b_gate,
           bn_gamma, bn_beta):
  src = edge_index[0]
  dst = edge_index[1]
  pad = E_PAD - E
  pad_dst = N + (jnp.arange(pad, dtype=jnp.int32) % (N_PAD - N))
  src_p = jnp.concatenate([src, jnp.zeros((pad,), jnp.int32)]).reshape(
      NC * NS * R, LW)
  dst_p = jnp.concatenate([dst, pad_dst]).reshape(NC * NS * R, LW)
  zerosN = jnp.zeros((N_PAD, H), jnp.float32)

  deg = _deg_call()(dst_p).reshape(NC, N_PAD)
  degA = deg[0, :N].reshape(N, 1)
  degB = deg[1, :N].reshape(N, 1)

  y = pl.pallas_call(
      _y_body,
      grid=(N // BLK,),
      in_specs=[_row_spec(D), _full_spec((H, D)), _row_spec(1),
                _row_spec(1)],
      out_specs=_row_spec(H),
      out_shape=jax.ShapeDtypeStruct((N, H), jnp.float32),
  )(x, W_conv, degA, degB)

  accs = _gather_scatter_call()(y, src_p, dst_p, zerosN)
  accA = accs[0, :N]
  accB = accs[1, :N]

  bc2 = b_conv.reshape(1, H)
  bg2 = b_gate.reshape(1, H)
  bl2 = b_lin.reshape(1, H)

  out_pre, s8, q8 = pl.pallas_call(
      _fuse_body,
      grid=(N // BLK,),
      in_specs=[_row_spec(H), _row_spec(H), _row_spec(H), _row_spec(1),
                _row_spec(1), _row_spec(D), _full_spec((H, H)),
                _full_spec((1, H)), _full_spec((H, D)), _full_spec((1, H)),
                _full_spec((1, H))],
      out_specs=[_row_spec(H), _full_spec((8, H)), _full_spec((8, H))],
      out_shape=[jax.ShapeDtypeStruct((N, H), jnp.float32),
                 jax.ShapeDtypeStruct((8, H), jnp.float32),
                 jax.ShapeDtypeStruct((8, H), jnp.float32)],
  )(accA, accB, y, degA, degB, x, W_gate, bg2, W_lin, bl2, bc2)

  out = pl.pallas_call(
      _bn_body,
      grid=(N // BLK,),
      in_specs=[_row_spec(H), _full_spec((8, H)), _full_spec((8, H)),
                _full_spec((1, H)), _full_spec((1, H))],
      out_specs=_row_spec(H),
      out_shape=jax.ShapeDtypeStruct((N, H), jnp.float32),
  )(out_pre, s8, q8, bn_gamma.reshape(1, H), bn_beta.reshape(1, H))
  return out
